# baseline (speedup 1.0000x reference)
import jax
import jax.numpy as jnp
from jax.experimental import pallas as pl

_HIDDEN = 2048
_NUM_EXPERTS = 16
_BLOCK = 512
_LENS = (6, 6, 5, 5, 5, 5)
_STEPS = 6
_STARTS = (0, 6, 12, 17, 22, 27)
_NS = len(_LENS)


def _route(x, wT, s_ref, p_ref):
    logits = jnp.dot(x, wT, preferred_element_type=jnp.float32)
    lmax = jnp.max(logits, axis=1, keepdims=True)
    e = jnp.exp(logits - lmax)
    scores = e * (1.0 / jnp.sum(e, axis=1, keepdims=True))
    s_ref[...] = scores

    iota = jax.lax.broadcasted_iota(jnp.int32, scores.shape, 1)
    bits = jax.lax.bitcast_convert_type(scores, jnp.int32)
    packed = jnp.bitwise_or(jnp.bitwise_and(bits, -16), 15 - iota)
    p1 = jnp.max(packed, axis=1, keepdims=True)
    p2 = jnp.max(jnp.where(packed == p1, jnp.int32(-2147483647 - 1), packed),
                 axis=1, keepdims=True)
    i1 = 15 - jnp.bitwise_and(p1, 15)
    i2 = 15 - jnp.bitwise_and(p2, 15)
    v1 = jax.lax.bitcast_convert_type(jnp.bitwise_and(p1, -16), jnp.float32)
    v2 = jax.lax.bitcast_convert_type(jnp.bitwise_and(p2, -16), jnp.float32)

    inv_norm = 1.0 / (v1 + v2)
    p_ref[...] = jnp.concatenate(
        [v1 * inv_norm, v2 * inv_norm,
         jax.lax.bitcast_convert_type(i1, jnp.float32),
         jax.lax.bitcast_convert_type(i2, jnp.float32)], axis=1)


def _router_block(*refs):
    x_refs = refs[:_NS]
    wT = refs[_NS][...]
    out_refs = refs[_NS + 1:]
    for k in range(_NS):
        _route(x_refs[k][...], wT, out_refs[2 * k], out_refs[2 * k + 1])


def kernel(x, W):
    n = x.shape[0]
    wT = W.T

    def in_map(k):
        s, ln = _STARTS[k], _LENS[k]
        return pl.BlockSpec((_BLOCK, _HIDDEN),
                            lambda i, s=s, ln=ln: (s + jnp.minimum(i, ln - 1), 0))

    def out_map(k):
        ln = _LENS[k]
        return lambda i, ln=ln: (jnp.minimum(i, ln - 1), 0)

    in_specs = [in_map(k) for k in range(_NS)]
    in_specs.append(pl.BlockSpec((_HIDDEN, _NUM_EXPERTS), lambda i: (0, 0)))

    out_specs = []
    out_shape = []
    for k in range(_NS):
        rows = _LENS[k] * _BLOCK
        out_specs.append(pl.BlockSpec((_BLOCK, _NUM_EXPERTS), out_map(k)))
        out_shape.append(jax.ShapeDtypeStruct((rows, _NUM_EXPERTS), jnp.float32))
        out_specs.append(pl.BlockSpec((_BLOCK, 4), out_map(k)))
        out_shape.append(jax.ShapeDtypeStruct((rows, 4), jnp.float32))

    outs = pl.pallas_call(
        _router_block,
        grid=(_STEPS,),
        in_specs=in_specs,
        out_specs=out_specs,
        out_shape=out_shape,
    )(*([x] * _NS), wT)

    scores = jnp.concatenate(outs[0::2], axis=0)
    packed = jnp.concatenate(outs[1::2], axis=0)
    ew = packed[:, :2]
    idx = jax.lax.bitcast_convert_type(packed[:, 2:4], jnp.int32)
    return (scores, ew, idx)


# speedup vs baseline: 1.1097x; 1.1097x over previous
import jax
import jax.numpy as jnp
from jax.experimental import pallas as pl

_HIDDEN = 2048
_NUM_EXPERTS = 16
_BLOCK = 512
_LENS = (8, 8, 8, 8)
_STEPS = 8
_STARTS = (0, 8, 16, 24)
_NS = len(_LENS)


def _route(x, wT, s_ref, p_ref):
    logits = jnp.dot(x, wT, preferred_element_type=jnp.float32)
    lmax = jnp.max(logits, axis=1, keepdims=True)
    e = jnp.exp(logits - lmax)
    scores = e * (1.0 / jnp.sum(e, axis=1, keepdims=True))
    s_ref[...] = scores

    iota = jax.lax.broadcasted_iota(jnp.int32, scores.shape, 1)
    bits = jax.lax.bitcast_convert_type(scores, jnp.int32)
    packed = jnp.bitwise_or(jnp.bitwise_and(bits, -16), 15 - iota)
    p1 = jnp.max(packed, axis=1, keepdims=True)
    p2 = jnp.max(jnp.where(packed == p1, jnp.int32(-2147483647 - 1), packed),
                 axis=1, keepdims=True)
    i1 = 15 - jnp.bitwise_and(p1, 15)
    i2 = 15 - jnp.bitwise_and(p2, 15)
    v1 = jax.lax.bitcast_convert_type(jnp.bitwise_and(p1, -16), jnp.float32)
    v2 = jax.lax.bitcast_convert_type(jnp.bitwise_and(p2, -16), jnp.float32)

    inv_norm = 1.0 / (v1 + v2)
    p_ref[...] = jnp.concatenate(
        [v1 * inv_norm, v2 * inv_norm,
         jax.lax.bitcast_convert_type(i1, jnp.float32),
         jax.lax.bitcast_convert_type(i2, jnp.float32)], axis=1)


def _router_block(*refs):
    x_refs = refs[:_NS]
    wT = refs[_NS][...]
    out_refs = refs[_NS + 1:]
    for k in range(_NS):
        _route(x_refs[k][...], wT, out_refs[2 * k], out_refs[2 * k + 1])


def kernel(x, W):
    n = x.shape[0]
    wT = W.T

    def in_map(k):
        s, ln = _STARTS[k], _LENS[k]
        return pl.BlockSpec((_BLOCK, _HIDDEN),
                            lambda i, s=s, ln=ln: (s + jnp.minimum(i, ln - 1), 0))

    def out_map(k):
        ln = _LENS[k]
        return lambda i, ln=ln: (jnp.minimum(i, ln - 1), 0)

    in_specs = [in_map(k) for k in range(_NS)]
    in_specs.append(pl.BlockSpec((_HIDDEN, _NUM_EXPERTS), lambda i: (0, 0)))

    out_specs = []
    out_shape = []
    for k in range(_NS):
        rows = _LENS[k] * _BLOCK
        out_specs.append(pl.BlockSpec((_BLOCK, _NUM_EXPERTS), out_map(k)))
        out_shape.append(jax.ShapeDtypeStruct((rows, _NUM_EXPERTS), jnp.float32))
        out_specs.append(pl.BlockSpec((_BLOCK, 4), out_map(k)))
        out_shape.append(jax.ShapeDtypeStruct((rows, 4), jnp.float32))

    outs = pl.pallas_call(
        _router_block,
        grid=(_STEPS,),
        in_specs=in_specs,
        out_specs=out_specs,
        out_shape=out_shape,
    )(*([x] * _NS), wT)

    scores = jnp.concatenate(outs[0::2], axis=0)
    packed = jnp.concatenate(outs[1::2], axis=0)
    ew = packed[:, :2]
    idx = jax.lax.bitcast_convert_type(packed[:, 2:4], jnp.int32)
    return (scores, ew, idx)
